# baseline (speedup 1.0000x reference)
import jax
import jax.numpy as jnp
from jax.experimental import pallas as pl
from jax.experimental.pallas import tpu as pltpu

HEADS = 12
DH = 64
E = 768
SEQ = 8192
ZOO = 256
TILE = 512
NTILES = SEQ // TILE
SCALE = 0.125


def _mmT(a, b):
    return jax.lax.dot_general(a, b, (((1,), (1,)), ((), ())),
                               preferred_element_type=jnp.float32)


def _mm(a, b):
    return jax.lax.dot_general(a, b, (((1,), (0,)), ((), ())),
                               preferred_element_type=jnp.float32)


def _fused_kernel(x_ref, mm_ref, topo_ref, func_ref, Wm_ref, bm_ref,
                  Wpm_ref, Wpt_ref, Wpf_ref, bp_ref, Win_ref, bin_ref,
                  hz_ref, Wg_ref, bg_ref,
                  Wd_ref, bd_ref,
                  Wo_ref, bo_ref, We_ref, be_ref,
                  me_ref, attn_ref, mean_ref, pred_ref,
                  kb_scr, vb_scr, q_scr, acc_scr, m_scr, l_scr, xs_scr):
    ph = pl.program_id(0)
    t = pl.program_id(1)

    @pl.when((ph == 0) & (t == 0))
    def _prologue():
        meta = jnp.maximum(_mmT(mm_ref[...], Wm_ref[...]) + bm_ref[...], 0.0)
        me = _mmT(meta, Wpm_ref[...])
        me += _mmT(topo_ref[...], Wpt_ref[...])
        me += _mmT(func_ref[...], Wpf_ref[...])
        me = jnp.maximum(me + bp_ref[...], 0.0)
        me_ref[...] = me
        q_scr[...] = _mmT(me, Win_ref[0:E, :]) + bin_ref[0:1, :]
        acc_scr[...] = jnp.zeros_like(acc_scr)
        m_scr[...] = jnp.full_like(m_scr, -jnp.inf)
        l_scr[...] = jnp.zeros_like(l_scr)
        xs_scr[...] = jnp.zeros_like(xs_scr)

    @pl.when(ph == 0)
    def _pass_kv():
        x = x_ref[...]
        de = _mmT(x, Wd_ref[...]) + bd_ref[...]
        xs_scr[...] += jnp.sum(de, axis=0, keepdims=True)
        k16 = (_mmT(de, Win_ref[E:2 * E, :]) + bin_ref[1:2, :]
               ).astype(jnp.bfloat16)
        v16 = (_mmT(de, Win_ref[2 * E:3 * E, :]) + bin_ref[2:3, :]
               ).astype(jnp.bfloat16)
        kb_scr[t] = k16
        vb_scr[t] = v16
        for h in range(HEADS):
            sl = slice(h * DH, (h + 1) * DH)
            s = _mmT(q_scr[:, sl], k16[:, sl]) * SCALE
            m_scr[:, h:h + 1] = jnp.maximum(
                m_scr[:, h:h + 1], jnp.max(s, axis=1, keepdims=True))

    @pl.when(ph == 1)
    def _pass_av():
        k16 = kb_scr[t]
        v16 = vb_scr[t]
        for h in range(HEADS):
            sl = slice(h * DH, (h + 1) * DH)
            s = _mmT(q_scr[:, sl], k16[:, sl]) * SCALE
            p = jnp.exp(s - m_scr[:, h:h + 1])
            l_scr[:, h:h + 1] += jnp.sum(p, axis=1, keepdims=True)
            acc_scr[:, sl] += _mm(p, v16[:, sl])

    @pl.when((ph == 1) & (t == NTILES - 1))
    def _epilogue():
        cols = []
        for h in range(HEADS):
            sl = slice(h * DH, (h + 1) * DH)
            cols.append(acc_scr[:, sl] / l_scr[:, h:h + 1])
        o = jnp.concatenate(cols, axis=1)
        attn = _mmT(o, Wo_ref[...]) + bo_ref[...]
        attn_ref[...] = attn
        mean_ref[...] = xs_scr[...] * (1.0 / SEQ)
        logits = hz_ref[...] * (1.0 / 720.0) * Wg_ref[...] + bg_ref[...]
        mx = jnp.max(logits, axis=1, keepdims=True)
        ex = jnp.exp(logits - mx)
        gate = ex / jnp.sum(ex, axis=1, keepdims=True)
        eo = _mmT(attn, We_ref[...]) + be_ref[...]
        pred_ref[...] = jnp.sum(eo * gate, axis=1, keepdims=True)


def kernel(x, m_meta_emb, d_meta_emb, topo_emb, func_emb, horizon,
           W_data, b_data, W_meta, b_meta, W_proj, b_proj,
           W_in, b_in, W_out, b_out, W_exp, b_exp, W_gate, b_gate):
    f32 = jnp.float32
    mm = m_meta_emb[0]
    topo = topo_emb[0]
    func = func_emb[0]
    x2 = x[0]
    META_IN = mm.shape[1]
    FUNC = func.shape[1]
    TOPO = topo.shape[1]
    MO = W_meta.shape[0]

    mm_p = jnp.pad(mm, ((0, 0), (0, 128 - META_IN)))
    Wm_p = jnp.pad(W_meta, ((0, 0), (0, 128 - META_IN)))
    func_p = jnp.pad(func, ((0, 0), (0, 128 - FUNC)))
    Wpm = W_proj[:, :MO]
    Wpt = W_proj[:, MO:MO + TOPO]
    Wpf = jnp.pad(W_proj[:, MO + TOPO:], ((0, 0), (0, 128 - FUNC)))

    row = lambda a: a.reshape(1, -1).astype(f32)

    const = lambda shp: pl.BlockSpec(shp, lambda p, t: tuple(0 for _ in shp))
    fused = pl.pallas_call(
        _fused_kernel,
        grid=(2, NTILES),
        in_specs=[
            pl.BlockSpec((TILE, E), lambda p, t: (jnp.where(p == 0, t, 0), 0)),
            const((ZOO, 128)), const((ZOO, 128)), const((ZOO, 128)),
            const((MO, 128)), const((1, MO)),
            const((E, MO)), const((E, TOPO)), const((E, 128)),
            const((1, E)), const((3 * E, E)), const((3, E)),
            const((1, 1)), const((1, 8)), const((1, 8)),
            const((E, E)), const((1, E)),
            const((E, E)), const((1, E)), const((8, E)), const((1, 8)),
        ],
        out_specs=[
            const((ZOO, E)), const((ZOO, E)), const((1, E)),
            const((ZOO, 1)),
        ],
        out_shape=[
            jax.ShapeDtypeStruct((ZOO, E), f32),
            jax.ShapeDtypeStruct((ZOO, E), f32),
            jax.ShapeDtypeStruct((1, E), f32),
            jax.ShapeDtypeStruct((ZOO, 1), f32),
        ],
        scratch_shapes=[
            pltpu.VMEM((NTILES, TILE, E), jnp.bfloat16),
            pltpu.VMEM((NTILES, TILE, E), jnp.bfloat16),
            pltpu.VMEM((ZOO, E), f32),
            pltpu.VMEM((ZOO, E), f32),
            pltpu.VMEM((ZOO, HEADS), f32),
            pltpu.VMEM((ZOO, HEADS), f32),
            pltpu.VMEM((1, E), f32),
        ],
    )
    model_emb, attn_out, mean_embed, pred = fused(
        x2, mm_p, topo, func_p, Wm_p, row(b_meta),
        Wpm, Wpt, Wpf, row(b_proj), W_in, b_in.reshape(3, E),
        horizon.reshape(1, 1), W_gate.reshape(1, -1), row(b_gate),
        W_data, row(b_data),
        W_out, row(b_out), W_exp, row(b_exp))

    prediction = pred.reshape(1, 1, ZOO)
    return (prediction, mean_embed, model_emb[None], attn_out[None])
